# baseline (speedup 1.0000x reference)
import jax
import jax.numpy as jnp
from jax import lax
from jax.experimental import pallas as pl
from jax.experimental.pallas import tpu as pltpu
from jax.experimental.pallas import tpu_sc as plsc


_NC = 2
_NS = 16
_NW = _NC * _NS

_I_MAIN = 31232
_I_CHUNK = 488
_BUF_ROWS = 512


def _sc_chunks(total, size):
    out = []
    off = 0
    while off < total:
        n = min(size, total - off)
        out.append((off, n))
        off += n
    return out


def _sc_stream_copy(transfers, bufs, sems_in, sems_out):
    outs = []
    for g, (src, dst, rows) in enumerate(transfers):
        b = g % 2
        if g >= 2:
            outs[g - 2].wait()
        buf = bufs[b].at[pl.ds(0, rows)]
        cin = pltpu.make_async_copy(src, buf, sems_in[b])
        cin.start()
        cin.wait()
        cout = pltpu.make_async_copy(buf, dst, sems_out[b])
        cout.start()
        outs.append(cout)
    for c in outs[-2:]:
        c.wait()


def _sc_transfers(src, dst, base, rows, chunk):
    return [
        (src.at[pl.ds(base + off, n)], dst.at[pl.ds(base + off, n)], n)
        for off, n in _sc_chunks(rows, chunk)
    ]


def _sc_body(i_in, i_out, buf0, buf1, si0, si1, so0, so1):
    wid = lax.axis_index("s") * _NC + lax.axis_index("c")
    bufs = (buf0, buf1)
    sems_in = (si0, si1)
    sems_out = (so0, so1)
    base = pl.multiple_of(wid * _I_MAIN, 8)
    _sc_stream_copy(
        _sc_transfers(i_in, i_out, base, _I_MAIN, _I_CHUNK),
        bufs, sems_in, sems_out,
    )

    tail_base = _I_MAIN * _NW

    @pl.when(wid == 0)
    def _():
        _sc_stream_copy(
            _sc_transfers(i_in, i_out, tail_base,
                          i_in.shape[0] - tail_base, _BUF_ROWS),
            bufs, sems_in, sems_out,
        )


def _sc_copy(item_weight):
    mesh = plsc.VectorSubcoreMesh(core_axis_name="c", subcore_axis_name="s")
    run = pl.kernel(
        _sc_body,
        out_type=jax.ShapeDtypeStruct(item_weight.shape, item_weight.dtype),
        mesh=mesh,
        scratch_types=[
            pltpu.VMEM((_BUF_ROWS, 32), jnp.float32),
            pltpu.VMEM((_BUF_ROWS, 32), jnp.float32),
            pltpu.SemaphoreType.DMA,
            pltpu.SemaphoreType.DMA,
            pltpu.SemaphoreType.DMA,
            pltpu.SemaphoreType.DMA,
        ],
    )
    return run(item_weight)


_NBUF = 6
_TC_CHUNK = 12800


def _tc_ring_copy(transfers, bufs, sems_in, sems_out):
    n = len(transfers)
    ins = [None] * n
    outs = [None] * n

    def start_in(g):
        src, _, rows = transfers[g]
        b = g % _NBUF
        ins[g] = pltpu.make_async_copy(src, bufs[b].at[pl.ds(0, rows)],
                                       sems_in[b])
        ins[g].start()

    for g in range(min(_NBUF, n)):
        start_in(g)
    for g in range(n):
        b = g % _NBUF
        _, dst, rows = transfers[g]
        ins[g].wait()
        outs[g] = pltpu.make_async_copy(bufs[b].at[pl.ds(0, rows)], dst,
                                        sems_out[b])
        outs[g].start()
        nxt = g + _NBUF
        if nxt < n:
            outs[g].wait()
            start_in(nxt)
    for g in range(max(0, n - _NBUF), n):
        outs[g].wait()


def _tc_body(u_in, u_out, *scratch):
    bufs = scratch[:_NBUF]
    sems_in = [scratch[_NBUF].at[k] for k in range(_NBUF)]
    sems_out = [scratch[_NBUF + 1].at[k] for k in range(_NBUF)]
    transfers = [
        (u_in.at[pl.ds(off, n)], u_out.at[pl.ds(off, n)], n)
        for off, n in _sc_chunks(u_in.shape[0], _TC_CHUNK)
    ]
    _tc_ring_copy(transfers, bufs, sems_in, sems_out)


def _tc_copy(user_weight):
    return pl.pallas_call(
        _tc_body,
        in_specs=[pl.BlockSpec(memory_space=pl.ANY)],
        out_specs=pl.BlockSpec(memory_space=pl.ANY),
        out_shape=jax.ShapeDtypeStruct(user_weight.shape, user_weight.dtype),
        scratch_shapes=[pltpu.VMEM((_TC_CHUNK, 32), jnp.float32)] * _NBUF
        + [pltpu.SemaphoreType.DMA((_NBUF,)),
           pltpu.SemaphoreType.DMA((_NBUF,))],
    )(user_weight)


def kernel(user_weight, item_weight, edge_index):
    i_out = _sc_copy(item_weight)
    u_out = _tc_copy(user_weight)
    return (u_out, i_out)
